# Initial kernel scaffold; baseline (speedup 1.0000x reference)
import jax
import jax.numpy as jnp
from jax.experimental import pallas as pl


def kernel(x, D, H, centroids, boundaries):
    raise NotImplementedError("write your pallas kernel here")


# speedup vs baseline: 9.1324x; 9.1324x over previous
import jax
import jax.numpy as jnp
from jax.experimental import pallas as pl
from jax.experimental.pallas import tpu as pltpu

_DIM = 64
_LEVELS = 8
_EPS = 1e-8
_BLOCK = 1024


def _pq_block(x_ref, d_ref, h_ref, cb_ref, cc_ref, ls_ref, o_ref):
    x = x_ref[...]
    d = d_ref[...]
    h = h_ref[...]

    y = jax.lax.dot_general(
        x * d, h, (((1,), (1,)), ((), ())),
        preferred_element_type=jnp.float32)

    s = y * y
    jj = jax.lax.broadcasted_iota(jnp.int32, (_DIM, _DIM), 0)
    ii = jax.lax.broadcasted_iota(jnp.int32, (_DIM, _DIM), 1)
    m_suffix = (jj >= ii).astype(jnp.float32)
    suffix2 = jax.lax.dot_general(
        s, m_suffix, (((1,), (0,)), ((), ())),
        preferred_element_type=jnp.float32,
        precision=jax.lax.Precision.HIGHEST)
    rem = jnp.sqrt(suffix2) + _EPS
    r = rem[:, 0:1]

    ct = jnp.clip(y / rem, -1.0 + _EPS, 1.0 - _EPS)

    cos_q = jnp.full_like(ct, cc_ref[0, 0])
    logsin_q = jnp.full_like(ct, ls_ref[0, 0])
    for k in range(1, _LEVELS):
        mask = ct < cb_ref[0, k - 1]
        cos_q = jnp.where(mask, cc_ref[0, k], cos_q)
        logsin_q = jnp.where(mask, ls_ref[0, k], logsin_q)

    col = jax.lax.broadcasted_iota(jnp.int32, cos_q.shape, 1)
    cos_q = jnp.where(col == _DIM - 1, 1.0, cos_q)

    m_excl = (jj < ii).astype(jnp.float32)
    lcs = jax.lax.dot_general(
        logsin_q, m_excl, (((1,), (0,)), ((), ())),
        preferred_element_type=jnp.float32,
        precision=jax.lax.Precision.HIGHEST)
    xp = r * jnp.exp(lcs) * cos_q

    out = jax.lax.dot_general(
        xp, h, (((1,), (0,)), ((), ())),
        preferred_element_type=jnp.float32)
    o_ref[...] = out * d


def kernel(x, D, H, centroids, boundaries):
    n = x.shape[0]
    d2 = D.reshape(1, _DIM).astype(jnp.float32)
    h = H.astype(jnp.float32)
    cos_b = jnp.cos(boundaries[1:_LEVELS]).reshape(1, _LEVELS - 1)
    cos_c = jnp.cos(centroids).reshape(1, _LEVELS)
    logsin_c = jnp.log(jnp.sin(centroids)).reshape(1, _LEVELS)

    blk = min(_BLOCK, n)
    pad = (-n) % blk
    xp = jnp.pad(x, ((0, pad), (0, 0))) if pad else x
    npad = n + pad

    out = pl.pallas_call(
        _pq_block,
        grid=(npad // blk,),
        in_specs=[
            pl.BlockSpec((blk, _DIM), lambda g: (g, 0)),
            pl.BlockSpec((1, _DIM), lambda g: (0, 0)),
            pl.BlockSpec((_DIM, _DIM), lambda g: (0, 0)),
            pl.BlockSpec(memory_space=pltpu.SMEM),
            pl.BlockSpec(memory_space=pltpu.SMEM),
            pl.BlockSpec(memory_space=pltpu.SMEM),
        ],
        out_specs=pl.BlockSpec((blk, _DIM), lambda g: (g, 0)),
        out_shape=jax.ShapeDtypeStruct((npad, _DIM), jnp.float32),
        compiler_params=pltpu.CompilerParams(
            dimension_semantics=("parallel",)),
    )(xp, d2, h, cos_b, cos_c, logsin_c)
    return out[:n] if pad else out


# speedup vs baseline: 19.4003x; 2.1243x over previous
import jax
import jax.numpy as jnp
from jax.experimental import pallas as pl
from jax.experimental.pallas import tpu as pltpu

_DIM = 64
_LEVELS = 8
_EPS = 1e-8
_BLOCK = 4096


def _pq_block(x_ref, d_ref, h_ref, cb_ref, cc_ref, ls_ref, o_ref):
    x = x_ref[...]
    d = d_ref[...]
    h = h_ref[...]

    y = jax.lax.dot_general(
        x * d, h, (((1,), (1,)), ((), ())),
        preferred_element_type=jnp.float32)

    s = y * y
    jj = jax.lax.broadcasted_iota(jnp.int32, (_DIM, _DIM), 0)
    ii = jax.lax.broadcasted_iota(jnp.int32, (_DIM, _DIM), 1)
    m_suffix = (jj >= ii).astype(jnp.float32)
    suffix2 = jax.lax.dot_general(
        s, m_suffix, (((1,), (0,)), ((), ())),
        preferred_element_type=jnp.float32)
    rem = jnp.sqrt(suffix2) + _EPS
    r = rem[:, 0:1]

    ct = jnp.clip(y / rem, -1.0 + _EPS, 1.0 - _EPS)

    cos_q = jnp.full_like(ct, cc_ref[0, 0])
    logsin_q = jnp.full_like(ct, ls_ref[0, 0])
    for k in range(1, _LEVELS):
        mask = ct < cb_ref[0, k - 1]
        cos_q = jnp.where(mask, cc_ref[0, k], cos_q)
        logsin_q = jnp.where(mask, ls_ref[0, k], logsin_q)

    col = jax.lax.broadcasted_iota(jnp.int32, cos_q.shape, 1)
    cos_q = jnp.where(col == _DIM - 1, 1.0, cos_q)

    m_excl = (jj < ii).astype(jnp.float32)
    lcs = jax.lax.dot_general(
        logsin_q, m_excl, (((1,), (0,)), ((), ())),
        preferred_element_type=jnp.float32)
    xp = r * jnp.exp(lcs) * cos_q

    out = jax.lax.dot_general(
        xp, h, (((1,), (0,)), ((), ())),
        preferred_element_type=jnp.float32)
    o_ref[...] = out * d


def kernel(x, D, H, centroids, boundaries):
    n = x.shape[0]
    d2 = D.reshape(1, _DIM).astype(jnp.float32)
    h = H.astype(jnp.float32)
    cos_b = jnp.cos(boundaries[1:_LEVELS]).reshape(1, _LEVELS - 1)
    cos_c = jnp.cos(centroids).reshape(1, _LEVELS)
    logsin_c = jnp.log(jnp.sin(centroids)).reshape(1, _LEVELS)

    blk = min(_BLOCK, n)
    pad = (-n) % blk
    xp = jnp.pad(x, ((0, pad), (0, 0))) if pad else x
    npad = n + pad

    out = pl.pallas_call(
        _pq_block,
        grid=(npad // blk,),
        in_specs=[
            pl.BlockSpec((blk, _DIM), lambda g: (g, 0)),
            pl.BlockSpec((1, _DIM), lambda g: (0, 0)),
            pl.BlockSpec((_DIM, _DIM), lambda g: (0, 0)),
            pl.BlockSpec(memory_space=pltpu.SMEM),
            pl.BlockSpec(memory_space=pltpu.SMEM),
            pl.BlockSpec(memory_space=pltpu.SMEM),
        ],
        out_specs=pl.BlockSpec((blk, _DIM), lambda g: (g, 0)),
        out_shape=jax.ShapeDtypeStruct((npad, _DIM), jnp.float32),
        compiler_params=pltpu.CompilerParams(
            dimension_semantics=("parallel",)),
    )(xp, d2, h, cos_b, cos_c, logsin_c)
    return out[:n] if pad else out
